# baseline (speedup 1.0000x reference)
import functools

import jax
import jax.numpy as jnp
from jax import lax
from jax.experimental import pallas as pl
from jax.experimental.pallas import tpu as pltpu
from jax.experimental.pallas import tpu_sc as plsc

N_NODES = 10000
N_EDGES = 320000
D = 128

NC = 2
NS = 16
NW = NC * NS

CHUNK = 128
CHUNKS_PER_TILE = 80
GRP = 8
EPT = CHUNK * CHUNKS_PER_TILE
E_PAD = EPT * NW
N_PAD = 10112
RPT = N_PAD // NS

_mesh = plsc.VectorSubcoreMesh(core_axis_name="c", subcore_axis_name="s")


def _make_agg_kernel():
    out_type = [jax.ShapeDtypeStruct((NC, N_PAD, D), jnp.float32)]
    scratch = [
        pltpu.VMEM_SHARED((N_PAD, D), jnp.float32),
        pltpu.VMEM((GRP, CHUNK), jnp.int32),
        pltpu.VMEM((GRP, CHUNK), jnp.int32),
        pltpu.VMEM((CHUNK, D), jnp.float32),
        pltpu.SemaphoreType.DMA,
    ]

    def body(h_hbm, src_hbm, dst_hbm, zagg_hbm, agg_out,
             agg_sh, src_v, dst_v, rows_v, sem):
        c = lax.axis_index("c")
        s = lax.axis_index("s")
        wid = c * NS + s

        pltpu.sync_copy(zagg_hbm.at[pl.ds(s * RPT, RPT)],
                        agg_sh.at[pl.ds(s * RPT, RPT)])
        plsc.subcore_barrier()

        def grp_body(g, carry):
            base = wid * CHUNKS_PER_TILE + g * GRP
            pltpu.sync_copy(src_hbm.at[pl.ds(base, GRP)], src_v)
            pltpu.sync_copy(dst_hbm.at[pl.ds(base, GRP)], dst_v)

            def chunk_body(j, c2):
                pltpu.async_copy(h_hbm.at[src_v.at[j]], rows_v, sem).wait()
                pltpu.sync_copy(rows_v, agg_sh.at[dst_v.at[j]], add=True)
                return c2

            lax.fori_loop(0, GRP, chunk_body, 0)
            return carry

        lax.fori_loop(0, CHUNKS_PER_TILE // GRP, grp_body, 0)
        plsc.subcore_barrier()

        pltpu.sync_copy(agg_sh.at[pl.ds(s * RPT, RPT)],
                        agg_out.at[c, pl.ds(s * RPT, RPT)])

    return functools.partial(pl.kernel, mesh=_mesh, out_type=out_type,
                             scratch_types=scratch)(body)


_agg_kernel = _make_agg_kernel()


BR = 400
GRID = N_NODES // BR


def _dense_body(aggp, degp, h, wm, ws, b, out):
    p = aggp[0] + aggp[1]
    deg = jnp.maximum(degp[0] + degp[1], 1.0)
    agg = p / deg
    out[...] = jnp.maximum(
        jnp.dot(agg, wm[...], preferred_element_type=jnp.float32)
        + jnp.dot(h[...], ws[...], preferred_element_type=jnp.float32)
        + b[...], 0.0)


def _dense_pool_body(aggp, degp, h, wm, ws, b, out):
    i = pl.program_id(0)
    p = aggp[0] + aggp[1]
    deg = jnp.maximum(degp[0] + degp[1], 1.0)
    agg = p / deg
    hn = jnp.maximum(
        jnp.dot(agg, wm[...], preferred_element_type=jnp.float32)
        + jnp.dot(h[...], ws[...], preferred_element_type=jnp.float32)
        + b[...], 0.0)
    part = jnp.sum(hn, axis=0, keepdims=True) * (1.0 / N_NODES)

    @pl.when(i == 0)
    def _init():
        out[...] = part

    @pl.when(i != 0)
    def _acc():
        out[...] = out[...] + part


_dense_specs = dict(
    grid=(GRID,),
    in_specs=[
        pl.BlockSpec((NC, BR, D), lambda i: (0, i, 0)),
        pl.BlockSpec((NC, BR, D), lambda i: (0, i, 0)),
        pl.BlockSpec((BR, D), lambda i: (i, 0)),
        pl.BlockSpec((D, D), lambda i: (0, 0)),
        pl.BlockSpec((D, D), lambda i: (0, 0)),
        pl.BlockSpec((1, D), lambda i: (0, 0)),
    ],
    compiler_params=pltpu.CompilerParams(
        dimension_semantics=("arbitrary",)),
)

_dense_layer = pl.pallas_call(
    _dense_body,
    out_shape=jax.ShapeDtypeStruct((N_NODES, D), jnp.float32),
    out_specs=pl.BlockSpec((BR, D), lambda i: (i, 0)),
    **_dense_specs,
)

_dense_pool_layer = pl.pallas_call(
    _dense_pool_body,
    out_shape=jax.ShapeDtypeStruct((1, D), jnp.float32),
    out_specs=pl.BlockSpec((1, D), lambda i: (0, 0)),
    **_dense_specs,
)


def kernel(x, edge_index, W_msg1, W_self1, b1, W_msg2, W_self2, b2):
    src = edge_index[0].astype(jnp.int32)
    dst = edge_index[1].astype(jnp.int32)
    pad = E_PAD - N_EDGES
    pad_dst = N_NODES + (jnp.arange(pad, dtype=jnp.int32) % (N_PAD - N_NODES))
    srcp = jnp.concatenate([src, jnp.zeros((pad,), jnp.int32)])
    dstp = jnp.concatenate([dst, pad_dst])
    srcp = srcp.reshape(E_PAD // CHUNK, CHUNK)
    dstp = dstp.reshape(E_PAD // CHUNK, CHUNK)
    zagg = jnp.zeros((N_PAD, D), jnp.float32)
    b1r = b1.reshape(1, D)
    b2r = b2.reshape(1, D)

    (aggp1,) = _agg_kernel(x, srcp, dstp, zagg)
    (degp,) = _agg_kernel(jnp.ones((N_NODES, D), jnp.float32), srcp, dstp, zagg)
    h1 = _dense_layer(aggp1, degp, x, W_msg1, W_self1, b1r)
    (aggp2,) = _agg_kernel(h1, srcp, dstp, zagg)
    out = _dense_pool_layer(aggp2, degp, h1, W_msg2, W_self2, b2r)
    return out


# speedup vs baseline: 1.1476x; 1.1476x over previous
import functools

import jax
import jax.numpy as jnp
from jax import lax
from jax.experimental import pallas as pl
from jax.experimental.pallas import tpu as pltpu
from jax.experimental.pallas import tpu_sc as plsc

N_NODES = 10000
N_EDGES = 320000
D = 128

NC = 2
NS = 16
NW = NC * NS

CHUNK = 128
CPT0 = 112
CPT1 = 48
GRP = 8
E_PAD = CHUNK * NS * (CPT0 + CPT1)
N_PAD = 10112
RPT = N_PAD // NS

_mesh = plsc.VectorSubcoreMesh(core_axis_name="c", subcore_axis_name="s")


def _make_agg_kernel():
    out_type = [jax.ShapeDtypeStruct((NC, N_PAD, D), jnp.float32)]
    scratch = [
        pltpu.VMEM_SHARED((N_PAD, D), jnp.float32),
        pltpu.VMEM((GRP, CHUNK), jnp.int32),
        pltpu.VMEM((GRP, CHUNK), jnp.int32),
        pltpu.VMEM((CHUNK, D), jnp.float32),
        pltpu.SemaphoreType.DMA,
    ]

    def body(h_hbm, src_hbm, dst_hbm, zagg_hbm, agg_out,
             agg_sh, src_v, dst_v, rows_v, sem):
        c = lax.axis_index("c")
        s = lax.axis_index("s")
        tile_base = jnp.where(c == 0, s * CPT0, NS * CPT0 + s * CPT1)
        n_grp = jnp.where(c == 0, CPT0 // GRP, CPT1 // GRP)

        pltpu.sync_copy(zagg_hbm.at[pl.ds(s * RPT, RPT)],
                        agg_sh.at[pl.ds(s * RPT, RPT)])
        plsc.subcore_barrier()

        def grp_body(g, carry):
            base = tile_base + g * GRP
            pltpu.sync_copy(src_hbm.at[pl.ds(base, GRP)], src_v)
            pltpu.sync_copy(dst_hbm.at[pl.ds(base, GRP)], dst_v)

            def chunk_body(j, c2):
                pltpu.async_copy(h_hbm.at[src_v.at[j]], rows_v, sem).wait()
                pltpu.sync_copy(rows_v, agg_sh.at[dst_v.at[j]], add=True)
                return c2

            lax.fori_loop(0, GRP, chunk_body, 0)
            return carry

        lax.fori_loop(0, n_grp, grp_body, 0)
        plsc.subcore_barrier()

        pltpu.sync_copy(agg_sh.at[pl.ds(s * RPT, RPT)],
                        agg_out.at[c, pl.ds(s * RPT, RPT)])

    return functools.partial(pl.kernel, mesh=_mesh, out_type=out_type,
                             scratch_types=scratch)(body)


_agg_kernel = _make_agg_kernel()


BR = 400
GRID = N_NODES // BR


def _dense_body(aggp, degp, h, wm, ws, b, out):
    p = aggp[0] + aggp[1]
    deg = jnp.maximum(degp[0] + degp[1], 1.0)
    agg = p / deg
    out[...] = jnp.maximum(
        jnp.dot(agg, wm[...], preferred_element_type=jnp.float32)
        + jnp.dot(h[...], ws[...], preferred_element_type=jnp.float32)
        + b[...], 0.0)


def _dense_pool_body(aggp, degp, h, wm, ws, b, out):
    i = pl.program_id(0)
    p = aggp[0] + aggp[1]
    deg = jnp.maximum(degp[0] + degp[1], 1.0)
    agg = p / deg
    hn = jnp.maximum(
        jnp.dot(agg, wm[...], preferred_element_type=jnp.float32)
        + jnp.dot(h[...], ws[...], preferred_element_type=jnp.float32)
        + b[...], 0.0)
    part = jnp.sum(hn, axis=0, keepdims=True) * (1.0 / N_NODES)

    @pl.when(i == 0)
    def _init():
        out[...] = part

    @pl.when(i != 0)
    def _acc():
        out[...] = out[...] + part


_dense_specs = dict(
    grid=(GRID,),
    in_specs=[
        pl.BlockSpec((NC, BR, D), lambda i: (0, i, 0)),
        pl.BlockSpec((NC, BR, D), lambda i: (0, i, 0)),
        pl.BlockSpec((BR, D), lambda i: (i, 0)),
        pl.BlockSpec((D, D), lambda i: (0, 0)),
        pl.BlockSpec((D, D), lambda i: (0, 0)),
        pl.BlockSpec((1, D), lambda i: (0, 0)),
    ],
    compiler_params=pltpu.CompilerParams(
        dimension_semantics=("arbitrary",)),
)

_dense_layer = pl.pallas_call(
    _dense_body,
    out_shape=jax.ShapeDtypeStruct((N_NODES, D), jnp.float32),
    out_specs=pl.BlockSpec((BR, D), lambda i: (i, 0)),
    **_dense_specs,
)

_dense_pool_layer = pl.pallas_call(
    _dense_pool_body,
    out_shape=jax.ShapeDtypeStruct((1, D), jnp.float32),
    out_specs=pl.BlockSpec((1, D), lambda i: (0, 0)),
    **_dense_specs,
)


def kernel(x, edge_index, W_msg1, W_self1, b1, W_msg2, W_self2, b2):
    src = edge_index[0].astype(jnp.int32)
    dst = edge_index[1].astype(jnp.int32)
    pad = E_PAD - N_EDGES
    pad_dst = N_NODES + (jnp.arange(pad, dtype=jnp.int32) % (N_PAD - N_NODES))
    srcp = jnp.concatenate([src, jnp.zeros((pad,), jnp.int32)])
    dstp = jnp.concatenate([dst, pad_dst])
    srcp = srcp.reshape(E_PAD // CHUNK, CHUNK)
    dstp = dstp.reshape(E_PAD // CHUNK, CHUNK)
    zagg = jnp.zeros((N_PAD, D), jnp.float32)
    b1r = b1.reshape(1, D)
    b2r = b2.reshape(1, D)

    (aggp1,) = _agg_kernel(x, srcp, dstp, zagg)
    (degp,) = _agg_kernel(jnp.ones((N_NODES, D), jnp.float32), srcp, dstp, zagg)
    h1 = _dense_layer(aggp1, degp, x, W_msg1, W_self1, b1r)
    (aggp2,) = _agg_kernel(h1, srcp, dstp, zagg)
    out = _dense_pool_layer(aggp2, degp, h1, W_msg2, W_self2, b2r)
    return out


# speedup vs baseline: 1.3437x; 1.1708x over previous
import functools

import jax
import jax.numpy as jnp
from jax import lax
from jax.experimental import pallas as pl
from jax.experimental.pallas import tpu as pltpu
from jax.experimental.pallas import tpu_sc as plsc

N_NODES = 10000
N_EDGES = 320000
D = 128

NC = 2
NS = 16

CHUNK = 128
CPT0 = 112
CPT1 = 48
GRP = 8
E_PAD = CHUNK * NS * (CPT0 + CPT1)
N_PAD = 10112
RPT = N_PAD // NS

_mesh = plsc.VectorSubcoreMesh(core_axis_name="c", subcore_axis_name="s")


def _tile_layout(c, s):
    tile_base = jnp.where(c == 0, s * CPT0, NS * CPT0 + s * CPT1)
    n_grp = jnp.where(c == 0, CPT0 // GRP, CPT1 // GRP)
    return tile_base, n_grp


def _make_agg_kernel():
    out_type = [jax.ShapeDtypeStruct((NC, N_PAD, D), jnp.float32)]
    scratch = [
        pltpu.VMEM_SHARED((N_PAD, D), jnp.float32),
        pltpu.VMEM((GRP, CHUNK), jnp.int32),
        pltpu.VMEM((GRP, CHUNK), jnp.int32),
        pltpu.VMEM((CHUNK, D), jnp.float32),
        pltpu.VMEM((CHUNK, D), jnp.float32),
        pltpu.SemaphoreType.DMA,
        pltpu.SemaphoreType.DMA,
        pltpu.SemaphoreType.DMA,
        pltpu.SemaphoreType.DMA,
    ]

    def body(h_hbm, src_hbm, dst_hbm, zagg_hbm, agg_out,
             agg_sh, src_v, dst_v, rows0, rows1, sg0, sg1, ss0, ss1):
        c = lax.axis_index("c")
        s = lax.axis_index("s")
        tile_base, n_grp = _tile_layout(c, s)
        bufs = ((rows0, sg0, ss0), (rows1, sg1, ss1))

        pltpu.sync_copy(zagg_hbm.at[pl.ds(s * RPT, RPT)],
                        agg_sh.at[pl.ds(s * RPT, RPT)])
        plsc.subcore_barrier()

        def grp_body(g, carry):
            base = tile_base + g * GRP
            pltpu.sync_copy(src_hbm.at[pl.ds(base, GRP)], src_v)
            pltpu.sync_copy(dst_hbm.at[pl.ds(base, GRP)], dst_v)

            for b in range(2):
                rows, sg, _ = bufs[b]
                pltpu.async_copy(h_hbm.at[src_v.at[b]], rows, sg)

            for j in range(GRP):
                rows, sg, ss = bufs[j % 2]
                pltpu.make_async_copy(h_hbm.at[src_v.at[j]], rows, sg).wait()
                pltpu.async_copy(rows, agg_sh.at[dst_v.at[j]], ss, add=True)
                if j + 2 < GRP:
                    pltpu.make_async_copy(rows, agg_sh.at[dst_v.at[j]],
                                          ss).wait()
                    pltpu.async_copy(h_hbm.at[src_v.at[j + 2]], rows, sg)

            for b in range(2):
                rows, _, ss = bufs[b]
                pltpu.make_async_copy(rows, agg_sh.at[dst_v.at[b]], ss).wait()
            return carry

        lax.fori_loop(0, n_grp, grp_body, 0)
        plsc.subcore_barrier()

        pltpu.sync_copy(agg_sh.at[pl.ds(s * RPT, RPT)],
                        agg_out.at[c, pl.ds(s * RPT, RPT)])

    return functools.partial(pl.kernel, mesh=_mesh, out_type=out_type,
                             scratch_types=scratch)(body)


def _make_deg_kernel():
    out_type = [jax.ShapeDtypeStruct((NC, N_PAD, D), jnp.float32)]
    scratch = [
        pltpu.VMEM_SHARED((N_PAD, D), jnp.float32),
        pltpu.VMEM((GRP, CHUNK), jnp.int32),
        pltpu.VMEM((CHUNK, D), jnp.float32),
        pltpu.SemaphoreType.DMA,
    ]

    def body(dst_hbm, zagg_hbm, ones_hbm, deg_out, deg_sh, dst_v, ones_v, ss):
        c = lax.axis_index("c")
        s = lax.axis_index("s")
        tile_base, n_grp = _tile_layout(c, s)

        pltpu.sync_copy(zagg_hbm.at[pl.ds(s * RPT, RPT)],
                        deg_sh.at[pl.ds(s * RPT, RPT)])
        pltpu.sync_copy(ones_hbm, ones_v)
        plsc.subcore_barrier()

        def grp_body(g, carry):
            pltpu.sync_copy(dst_hbm.at[pl.ds(tile_base + g * GRP, GRP)], dst_v)
            for j in range(GRP):
                pltpu.async_copy(ones_v, deg_sh.at[dst_v.at[j]], ss, add=True)
            for j in range(GRP):
                pltpu.make_async_copy(ones_v, deg_sh.at[dst_v.at[j]],
                                      ss).wait()
            return carry

        lax.fori_loop(0, n_grp, grp_body, 0)
        plsc.subcore_barrier()

        pltpu.sync_copy(deg_sh.at[pl.ds(s * RPT, RPT)],
                        deg_out.at[c, pl.ds(s * RPT, RPT)])

    return functools.partial(pl.kernel, mesh=_mesh, out_type=out_type,
                             scratch_types=scratch)(body)


_agg_kernel = _make_agg_kernel()
_deg_kernel = _make_deg_kernel()


BR = 400
GRID = N_NODES // BR


def _affine(aggp, degp, h, wm, ws, b):
    p = aggp[0] + aggp[1]
    deg = jnp.maximum(degp[0] + degp[1], 1.0)
    agg = p / deg
    return jnp.maximum(
        jnp.dot(agg, wm[...], preferred_element_type=jnp.float32)
        + jnp.dot(h[...], ws[...], preferred_element_type=jnp.float32)
        + b[...], 0.0)


def _dense_body(aggp, degp, h, wm, ws, b, out):
    out[...] = _affine(aggp, degp, h, wm, ws, b)


def _dense_pool_body(aggp, degp, h, wm, ws, b, out):
    i = pl.program_id(0)
    hn = _affine(aggp, degp, h, wm, ws, b)
    part = jnp.sum(hn, axis=0, keepdims=True) * (1.0 / N_NODES)

    @pl.when(i == 0)
    def _init():
        out[...] = part

    @pl.when(i != 0)
    def _acc():
        out[...] = out[...] + part


_dense_specs = dict(
    grid=(GRID,),
    in_specs=[
        pl.BlockSpec((NC, BR, D), lambda i: (0, i, 0)),
        pl.BlockSpec((NC, BR, D), lambda i: (0, i, 0)),
        pl.BlockSpec((BR, D), lambda i: (i, 0)),
        pl.BlockSpec((D, D), lambda i: (0, 0)),
        pl.BlockSpec((D, D), lambda i: (0, 0)),
        pl.BlockSpec((1, D), lambda i: (0, 0)),
    ],
    compiler_params=pltpu.CompilerParams(
        dimension_semantics=("arbitrary",)),
)

_dense_layer = pl.pallas_call(
    _dense_body,
    out_shape=jax.ShapeDtypeStruct((N_NODES, D), jnp.float32),
    out_specs=pl.BlockSpec((BR, D), lambda i: (i, 0)),
    **_dense_specs,
)

_dense_pool_layer = pl.pallas_call(
    _dense_pool_body,
    out_shape=jax.ShapeDtypeStruct((1, D), jnp.float32),
    out_specs=pl.BlockSpec((1, D), lambda i: (0, 0)),
    **_dense_specs,
)


def kernel(x, edge_index, W_msg1, W_self1, b1, W_msg2, W_self2, b2):
    src = edge_index[0].astype(jnp.int32)
    dst = edge_index[1].astype(jnp.int32)
    pad = E_PAD - N_EDGES
    pad_dst = N_NODES + (jnp.arange(pad, dtype=jnp.int32) % (N_PAD - N_NODES))
    srcp = jnp.concatenate([src, jnp.zeros((pad,), jnp.int32)])
    dstp = jnp.concatenate([dst, pad_dst])
    srcp = srcp.reshape(E_PAD // CHUNK, CHUNK)
    dstp = dstp.reshape(E_PAD // CHUNK, CHUNK)
    zagg = jnp.zeros((N_PAD, D), jnp.float32)
    ones = jnp.ones((CHUNK, D), jnp.float32)
    b1r = b1.reshape(1, D)
    b2r = b2.reshape(1, D)

    (aggp1,) = _agg_kernel(x, srcp, dstp, zagg)
    (degp,) = _deg_kernel(dstp, zagg, ones)
    h1 = _dense_layer(aggp1, degp, x, W_msg1, W_self1, b1r)
    (aggp2,) = _agg_kernel(h1, srcp, dstp, zagg)
    out = _dense_pool_layer(aggp2, degp, h1, W_msg2, W_self2, b2r)
    return out


# speedup vs baseline: 1.3614x; 1.0132x over previous
import functools

import jax
import jax.numpy as jnp
from jax import lax
from jax.experimental import pallas as pl
from jax.experimental.pallas import tpu as pltpu
from jax.experimental.pallas import tpu_sc as plsc

N_NODES = 10000
N_EDGES = 320000
D = 128

NC = 2
NS = 16

CHUNK = 128
CPT = 160
GRP = 8
N_GRP = CPT // GRP
E_PAD = CHUNK * NS * CPT
N_PAD = 10112
RPT = N_PAD // NS

_mesh = plsc.VectorSubcoreMesh(core_axis_name="c", subcore_axis_name="s")


def _zero_stripe(zagg_hbm, sh, s):
    pltpu.sync_copy(zagg_hbm.at[pl.ds(s * RPT, RPT)],
                    sh.at[pl.ds(s * RPT, RPT)])


def _publish_stripe(sh, out, s):
    pltpu.sync_copy(sh.at[pl.ds(s * RPT, RPT)], out.at[pl.ds(s * RPT, RPT)])


def _gather_scatter_groups(h_hbm, src_hbm, dst_hbm, acc_sh,
                           src_v, dst_v, bufs, s):
    def grp_body(g, carry):
        base = s * CPT + g * GRP
        pltpu.sync_copy(src_hbm.at[pl.ds(base, GRP)], src_v)
        pltpu.sync_copy(dst_hbm.at[pl.ds(base, GRP)], dst_v)

        for b in range(2):
            rows, sg, _ = bufs[b]
            pltpu.async_copy(h_hbm.at[src_v.at[b]], rows, sg)

        for j in range(GRP):
            rows, sg, ss = bufs[j % 2]
            pltpu.make_async_copy(h_hbm.at[src_v.at[j]], rows, sg).wait()
            pltpu.async_copy(rows, acc_sh.at[dst_v.at[j]], ss, add=True)
            if j + 2 < GRP:
                pltpu.make_async_copy(rows, acc_sh.at[dst_v.at[j]],
                                      ss).wait()
                pltpu.async_copy(h_hbm.at[src_v.at[j + 2]], rows, sg)

        for b in range(2):
            rows, _, ss = bufs[b]
            pltpu.make_async_copy(rows, acc_sh.at[dst_v.at[b]], ss).wait()
        return carry

    lax.fori_loop(0, N_GRP, grp_body, 0)


def _ones_scatter_groups(dst_hbm, acc_sh, dst_v, ones_v, ss, s):
    def grp_body(g, carry):
        pltpu.sync_copy(dst_hbm.at[pl.ds(s * CPT + g * GRP, GRP)], dst_v)
        for j in range(GRP):
            pltpu.async_copy(ones_v, acc_sh.at[dst_v.at[j]], ss, add=True)
        for j in range(GRP):
            pltpu.make_async_copy(ones_v, acc_sh.at[dst_v.at[j]], ss).wait()
        return carry

    lax.fori_loop(0, N_GRP, grp_body, 0)


def _common_scratch():
    return [
        pltpu.VMEM_SHARED((N_PAD, D), jnp.float32),
        pltpu.VMEM((GRP, CHUNK), jnp.int32),
        pltpu.VMEM((GRP, CHUNK), jnp.int32),
        pltpu.VMEM((CHUNK, D), jnp.float32),
        pltpu.VMEM((CHUNK, D), jnp.float32),
        pltpu.SemaphoreType.DMA,
        pltpu.SemaphoreType.DMA,
        pltpu.SemaphoreType.DMA,
        pltpu.SemaphoreType.DMA,
    ]


def _make_agg_deg_kernel():
    out_type = [jax.ShapeDtypeStruct((N_PAD, D), jnp.float32),
                jax.ShapeDtypeStruct((N_PAD, D), jnp.float32)]

    def body(h_hbm, src_hbm, dst_hbm, zagg_hbm, ones_hbm, agg_out, deg_out,
             acc_sh, src_v, dst_v, rows0, rows1, sg0, sg1, ss0, ss1):
        c = lax.axis_index("c")
        s = lax.axis_index("s")

        _zero_stripe(zagg_hbm, acc_sh, s)

        @pl.when(c == 1)
        def _load_ones():
            pltpu.sync_copy(ones_hbm, rows0)

        plsc.subcore_barrier()

        @pl.when(c == 0)
        def _agg():
            bufs = ((rows0, sg0, ss0), (rows1, sg1, ss1))
            _gather_scatter_groups(h_hbm, src_hbm, dst_hbm, acc_sh,
                                   src_v, dst_v, bufs, s)

        @pl.when(c == 1)
        def _deg():
            _ones_scatter_groups(dst_hbm, acc_sh, dst_v, rows0, ss0, s)

        plsc.subcore_barrier()

        @pl.when(c == 0)
        def _pub_agg():
            _publish_stripe(acc_sh, agg_out, s)

        @pl.when(c == 1)
        def _pub_deg():
            _publish_stripe(acc_sh, deg_out, s)

    return functools.partial(pl.kernel, mesh=_mesh, out_type=out_type,
                             scratch_types=_common_scratch())(body)


def _make_agg_kernel():
    out_type = [jax.ShapeDtypeStruct((N_PAD, D), jnp.float32)]

    def body(h_hbm, src_hbm, dst_hbm, zagg_hbm, agg_out,
             acc_sh, src_v, dst_v, rows0, rows1, sg0, sg1, ss0, ss1):
        c = lax.axis_index("c")
        s = lax.axis_index("s")

        @pl.when(c == 0)
        def _zero():
            _zero_stripe(zagg_hbm, acc_sh, s)

        plsc.subcore_barrier()

        @pl.when(c == 0)
        def _agg():
            bufs = ((rows0, sg0, ss0), (rows1, sg1, ss1))
            _gather_scatter_groups(h_hbm, src_hbm, dst_hbm, acc_sh,
                                   src_v, dst_v, bufs, s)

        plsc.subcore_barrier()

        @pl.when(c == 0)
        def _pub():
            _publish_stripe(acc_sh, agg_out, s)

    return functools.partial(pl.kernel, mesh=_mesh, out_type=out_type,
                             scratch_types=_common_scratch())(body)


_agg_deg_kernel = _make_agg_deg_kernel()
_agg_kernel = _make_agg_kernel()


BR = 400
GRID = N_NODES // BR


def _affine(agg_ref, deg_ref, h, wm, ws, b):
    deg = jnp.maximum(deg_ref[...], 1.0)
    agg = agg_ref[...] / deg
    return jnp.maximum(
        jnp.dot(agg, wm[...], preferred_element_type=jnp.float32)
        + jnp.dot(h[...], ws[...], preferred_element_type=jnp.float32)
        + b[...], 0.0)


def _dense_body(agg, deg, h, wm, ws, b, out):
    out[...] = _affine(agg, deg, h, wm, ws, b)


def _dense_pool_body(agg, deg, h, wm, ws, b, out):
    i = pl.program_id(0)
    hn = _affine(agg, deg, h, wm, ws, b)
    part = jnp.sum(hn, axis=0, keepdims=True) * (1.0 / N_NODES)

    @pl.when(i == 0)
    def _init():
        out[...] = part

    @pl.when(i != 0)
    def _acc():
        out[...] = out[...] + part


_dense_specs = dict(
    grid=(GRID,),
    in_specs=[
        pl.BlockSpec((BR, D), lambda i: (i, 0)),
        pl.BlockSpec((BR, D), lambda i: (i, 0)),
        pl.BlockSpec((BR, D), lambda i: (i, 0)),
        pl.BlockSpec((D, D), lambda i: (0, 0)),
        pl.BlockSpec((D, D), lambda i: (0, 0)),
        pl.BlockSpec((1, D), lambda i: (0, 0)),
    ],
    compiler_params=pltpu.CompilerParams(
        dimension_semantics=("arbitrary",)),
)

_dense_layer = pl.pallas_call(
    _dense_body,
    out_shape=jax.ShapeDtypeStruct((N_NODES, D), jnp.float32),
    out_specs=pl.BlockSpec((BR, D), lambda i: (i, 0)),
    **_dense_specs,
)

_dense_pool_layer = pl.pallas_call(
    _dense_pool_body,
    out_shape=jax.ShapeDtypeStruct((1, D), jnp.float32),
    out_specs=pl.BlockSpec((1, D), lambda i: (0, 0)),
    **_dense_specs,
)


def kernel(x, edge_index, W_msg1, W_self1, b1, W_msg2, W_self2, b2):
    src = edge_index[0].astype(jnp.int32)
    dst = edge_index[1].astype(jnp.int32)
    pad = E_PAD - N_EDGES
    pad_dst = N_NODES + (jnp.arange(pad, dtype=jnp.int32) % (N_PAD - N_NODES))
    srcp = jnp.concatenate([src, jnp.zeros((pad,), jnp.int32)])
    dstp = jnp.concatenate([dst, pad_dst])
    srcp = srcp.reshape(E_PAD // CHUNK, CHUNK)
    dstp = dstp.reshape(E_PAD // CHUNK, CHUNK)
    zagg = jnp.zeros((N_PAD, D), jnp.float32)
    ones = jnp.ones((CHUNK, D), jnp.float32)
    b1r = b1.reshape(1, D)
    b2r = b2.reshape(1, D)

    agg1, deg = _agg_deg_kernel(x, srcp, dstp, zagg, ones)
    h1 = _dense_layer(agg1, deg, x, W_msg1, W_self1, b1r)
    (agg2,) = _agg_kernel(h1, srcp, dstp, zagg)
    out = _dense_pool_layer(agg2, deg, h1, W_msg2, W_self2, b2r)
    return out


# speedup vs baseline: 1.4844x; 1.0904x over previous
import functools

import jax
import jax.numpy as jnp
from jax import lax
from jax.experimental import pallas as pl
from jax.experimental.pallas import tpu as pltpu
from jax.experimental.pallas import tpu_sc as plsc

N_NODES = 10000
N_EDGES = 320000
D = 128

NC = 2
NS = 16

CHUNK = 128
CPT = 160
GRP = 8
N_GRP = CPT // GRP
E_PAD = CHUNK * NS * CPT
N_PAD = 10112
RPT = N_PAD // NS

_mesh = plsc.VectorSubcoreMesh(core_axis_name="c", subcore_axis_name="s")


def _zero_stripe(zagg_hbm, sh, s):
    pltpu.sync_copy(zagg_hbm.at[pl.ds(s * RPT, RPT)],
                    sh.at[pl.ds(s * RPT, RPT)])


def _publish_stripe(sh, out, s):
    pltpu.sync_copy(sh.at[pl.ds(s * RPT, RPT)], out.at[pl.ds(s * RPT, RPT)])


def _gather_scatter_groups(h_hbm, src_hbm, dst_hbm, acc_sh,
                           src_v, dst_v, bufs, tile_base, n_grp):
    def grp_body(g, carry):
        base = tile_base + g * GRP
        pltpu.sync_copy(src_hbm.at[pl.ds(base, GRP)], src_v)
        pltpu.sync_copy(dst_hbm.at[pl.ds(base, GRP)], dst_v)

        for b in range(2):
            rows, sg, _ = bufs[b]
            pltpu.async_copy(h_hbm.at[src_v.at[b]], rows, sg)

        for j in range(GRP):
            rows, sg, ss = bufs[j % 2]
            pltpu.make_async_copy(h_hbm.at[src_v.at[j]], rows, sg).wait()
            pltpu.async_copy(rows, acc_sh.at[dst_v.at[j]], ss, add=True)
            if j + 2 < GRP:
                pltpu.make_async_copy(rows, acc_sh.at[dst_v.at[j]],
                                      ss).wait()
                pltpu.async_copy(h_hbm.at[src_v.at[j + 2]], rows, sg)

        for b in range(2):
            rows, _, ss = bufs[b]
            pltpu.make_async_copy(rows, acc_sh.at[dst_v.at[b]], ss).wait()
        return carry

    lax.fori_loop(0, n_grp, grp_body, 0)


def _ones_scatter_groups(dst_hbm, acc_sh, dst_v, ones_v, ss, s):
    def grp_body(g, carry):
        pltpu.sync_copy(dst_hbm.at[pl.ds(s * CPT + g * GRP, GRP)], dst_v)
        for j in range(GRP):
            pltpu.async_copy(ones_v, acc_sh.at[dst_v.at[j]], ss, add=True)
        for j in range(GRP):
            pltpu.make_async_copy(ones_v, acc_sh.at[dst_v.at[j]], ss).wait()
        return carry

    lax.fori_loop(0, N_GRP, grp_body, 0)


def _common_scratch():
    return [
        pltpu.VMEM_SHARED((N_PAD, D), jnp.float32),
        pltpu.VMEM((GRP, CHUNK), jnp.int32),
        pltpu.VMEM((GRP, CHUNK), jnp.int32),
        pltpu.VMEM((CHUNK, D), jnp.float32),
        pltpu.VMEM((CHUNK, D), jnp.float32),
        pltpu.SemaphoreType.DMA,
        pltpu.SemaphoreType.DMA,
        pltpu.SemaphoreType.DMA,
        pltpu.SemaphoreType.DMA,
    ]


def _make_agg_deg_kernel():
    out_type = [jax.ShapeDtypeStruct((N_PAD, D), jnp.float32),
                jax.ShapeDtypeStruct((N_PAD, D), jnp.float32)]

    def body(h_hbm, src_hbm, dst_hbm, zagg_hbm, ones_hbm, agg_out, deg_out,
             acc_sh, src_v, dst_v, rows0, rows1, sg0, sg1, ss0, ss1):
        c = lax.axis_index("c")
        s = lax.axis_index("s")

        _zero_stripe(zagg_hbm, acc_sh, s)

        @pl.when(c == 1)
        def _load_ones():
            pltpu.sync_copy(ones_hbm, rows0)

        plsc.subcore_barrier()

        @pl.when(c == 0)
        def _agg():
            bufs = ((rows0, sg0, ss0), (rows1, sg1, ss1))
            _gather_scatter_groups(h_hbm, src_hbm, dst_hbm, acc_sh,
                                   src_v, dst_v, bufs, s * CPT, N_GRP)

        @pl.when(c == 1)
        def _deg():
            _ones_scatter_groups(dst_hbm, acc_sh, dst_v, rows0, ss0, s)

        plsc.subcore_barrier()

        @pl.when(c == 0)
        def _pub_agg():
            _publish_stripe(acc_sh, agg_out, s)

        @pl.when(c == 1)
        def _pub_deg():
            _publish_stripe(acc_sh, deg_out, s)

    return functools.partial(pl.kernel, mesh=_mesh, out_type=out_type,
                             scratch_types=_common_scratch())(body)


CPT0 = 112
CPT1 = CPT - CPT0


def _make_agg_kernel():
    out_type = [jax.ShapeDtypeStruct((NC, N_PAD, D), jnp.float32)]

    def body(h_hbm, src_hbm, dst_hbm, zagg_hbm, agg_out,
             acc_sh, src_v, dst_v, rows0, rows1, sg0, sg1, ss0, ss1):
        c = lax.axis_index("c")
        s = lax.axis_index("s")
        tile_base = jnp.where(c == 0, s * CPT0, NS * CPT0 + s * CPT1)
        n_grp = jnp.where(c == 0, CPT0 // GRP, CPT1 // GRP)

        _zero_stripe(zagg_hbm, acc_sh, s)
        plsc.subcore_barrier()

        bufs = ((rows0, sg0, ss0), (rows1, sg1, ss1))
        _gather_scatter_groups(h_hbm, src_hbm, dst_hbm, acc_sh,
                               src_v, dst_v, bufs, tile_base, n_grp)

        plsc.subcore_barrier()
        pltpu.sync_copy(acc_sh.at[pl.ds(s * RPT, RPT)],
                        agg_out.at[c, pl.ds(s * RPT, RPT)])

    return functools.partial(pl.kernel, mesh=_mesh, out_type=out_type,
                             scratch_types=_common_scratch())(body)


_agg_deg_kernel = _make_agg_deg_kernel()
_agg_kernel = _make_agg_kernel()


BR = 400
GRID = N_NODES // BR


def _affine(agg_ref, deg_ref, h, wm, ws, b):
    deg = jnp.maximum(deg_ref[...], 1.0)
    agg = agg_ref[...] / deg
    return jnp.maximum(
        jnp.dot(agg, wm[...], preferred_element_type=jnp.float32)
        + jnp.dot(h[...], ws[...], preferred_element_type=jnp.float32)
        + b[...], 0.0)


def _dense_body(agg, deg, h, wm, ws, b, out):
    out[...] = _affine(agg, deg, h, wm, ws, b)


def _affine2(aggp, deg_ref, h, wm, ws, b):
    deg = jnp.maximum(deg_ref[...], 1.0)
    agg = (aggp[0] + aggp[1]) / deg
    return jnp.maximum(
        jnp.dot(agg, wm[...], preferred_element_type=jnp.float32)
        + jnp.dot(h[...], ws[...], preferred_element_type=jnp.float32)
        + b[...], 0.0)


def _dense_pool_body(aggp, deg, h, wm, ws, b, out):
    i = pl.program_id(0)
    hn = _affine2(aggp, deg, h, wm, ws, b)
    part = jnp.sum(hn, axis=0, keepdims=True) * (1.0 / N_NODES)

    @pl.when(i == 0)
    def _init():
        out[...] = part

    @pl.when(i != 0)
    def _acc():
        out[...] = out[...] + part


_dense_specs = dict(
    grid=(GRID,),
    in_specs=[
        pl.BlockSpec((BR, D), lambda i: (i, 0)),
        pl.BlockSpec((BR, D), lambda i: (i, 0)),
        pl.BlockSpec((BR, D), lambda i: (i, 0)),
        pl.BlockSpec((D, D), lambda i: (0, 0)),
        pl.BlockSpec((D, D), lambda i: (0, 0)),
        pl.BlockSpec((1, D), lambda i: (0, 0)),
    ],
    compiler_params=pltpu.CompilerParams(
        dimension_semantics=("arbitrary",)),
)

_dense_layer = pl.pallas_call(
    _dense_body,
    out_shape=jax.ShapeDtypeStruct((N_NODES, D), jnp.float32),
    out_specs=pl.BlockSpec((BR, D), lambda i: (i, 0)),
    **_dense_specs,
)

_dense_pool_specs = dict(_dense_specs)
_dense_pool_specs["in_specs"] = (
    [pl.BlockSpec((NC, BR, D), lambda i: (0, i, 0))]
    + list(_dense_specs["in_specs"][1:]))

_dense_pool_layer = pl.pallas_call(
    _dense_pool_body,
    out_shape=jax.ShapeDtypeStruct((1, D), jnp.float32),
    out_specs=pl.BlockSpec((1, D), lambda i: (0, 0)),
    **_dense_pool_specs,
)


def kernel(x, edge_index, W_msg1, W_self1, b1, W_msg2, W_self2, b2):
    src = edge_index[0].astype(jnp.int32)
    dst = edge_index[1].astype(jnp.int32)
    pad = E_PAD - N_EDGES
    pad_dst = N_NODES + (jnp.arange(pad, dtype=jnp.int32) % (N_PAD - N_NODES))
    srcp = jnp.concatenate([src, jnp.zeros((pad,), jnp.int32)])
    dstp = jnp.concatenate([dst, pad_dst])
    srcp = srcp.reshape(E_PAD // CHUNK, CHUNK)
    dstp = dstp.reshape(E_PAD // CHUNK, CHUNK)
    zagg = jnp.zeros((N_PAD, D), jnp.float32)
    ones = jnp.ones((CHUNK, D), jnp.float32)
    b1r = b1.reshape(1, D)
    b2r = b2.reshape(1, D)

    agg1, deg = _agg_deg_kernel(x, srcp, dstp, zagg, ones)
    h1 = _dense_layer(agg1, deg, x, W_msg1, W_self1, b1r)
    (agg2,) = _agg_kernel(h1, srcp, dstp, zagg)
    out = _dense_pool_layer(agg2, deg, h1, W_msg2, W_self2, b2r)
    return out


# speedup vs baseline: 1.5231x; 1.0261x over previous
import functools

import jax
import jax.numpy as jnp
from jax import lax
from jax.experimental import pallas as pl
from jax.experimental.pallas import tpu as pltpu
from jax.experimental.pallas import tpu_sc as plsc

N_NODES = 10000
N_EDGES = 320000
D = 128

NC = 2
NS = 16

CHUNK = 128
CPT = 160
GRP = 8
N_GRP = CPT // GRP
E_PAD = CHUNK * NS * CPT
N_PAD = 10112
RPT = N_PAD // NS

_mesh = plsc.VectorSubcoreMesh(core_axis_name="c", subcore_axis_name="s")


def _zero_stripe(zagg_hbm, sh, s):
    pltpu.sync_copy(zagg_hbm.at[pl.ds(s * RPT, RPT)],
                    sh.at[pl.ds(s * RPT, RPT)])


def _publish_stripe(sh, out, s):
    pltpu.sync_copy(sh.at[pl.ds(s * RPT, RPT)], out.at[pl.ds(s * RPT, RPT)])


def _gather_scatter_groups(h_hbm, src_hbm, dst_hbm, acc_sh,
                           src_v, dst_v, bufs, tile_base, n_grp):
    def grp_body(g, carry):
        base = tile_base + g * GRP
        pltpu.sync_copy(src_hbm.at[pl.ds(base, GRP)], src_v)
        pltpu.sync_copy(dst_hbm.at[pl.ds(base, GRP)], dst_v)

        for b in range(2):
            rows, sg, _ = bufs[b]
            pltpu.async_copy(h_hbm.at[src_v.at[b]], rows, sg)

        for j in range(GRP):
            rows, sg, ss = bufs[j % 2]
            pltpu.make_async_copy(h_hbm.at[src_v.at[j]], rows, sg).wait()
            pltpu.async_copy(rows, acc_sh.at[dst_v.at[j]], ss, add=True)
            if j + 2 < GRP:
                pltpu.make_async_copy(rows, acc_sh.at[dst_v.at[j]],
                                      ss).wait()
                pltpu.async_copy(h_hbm.at[src_v.at[j + 2]], rows, sg)

        for b in range(2):
            rows, _, ss = bufs[b]
            pltpu.make_async_copy(rows, acc_sh.at[dst_v.at[b]], ss).wait()
        return carry

    lax.fori_loop(0, n_grp, grp_body, 0)


def _ones_scatter_groups(dst_hbm, acc_sh, dst_v, ones_v, ss, s):
    def grp_body(g, carry):
        pltpu.sync_copy(dst_hbm.at[pl.ds(s * CPT + g * GRP, GRP)], dst_v)
        for j in range(GRP):
            pltpu.async_copy(ones_v, acc_sh.at[dst_v.at[j]], ss, add=True)
        for j in range(GRP):
            pltpu.make_async_copy(ones_v, acc_sh.at[dst_v.at[j]], ss).wait()
        return carry

    lax.fori_loop(0, N_GRP, grp_body, 0)


def _common_scratch():
    return [
        pltpu.VMEM_SHARED((N_PAD, D), jnp.float32),
        pltpu.VMEM((GRP, CHUNK), jnp.int32),
        pltpu.VMEM((GRP, CHUNK), jnp.int32),
        pltpu.VMEM((CHUNK, D), jnp.float32),
        pltpu.VMEM((CHUNK, D), jnp.float32),
        pltpu.SemaphoreType.DMA,
        pltpu.SemaphoreType.DMA,
        pltpu.SemaphoreType.DMA,
        pltpu.SemaphoreType.DMA,
    ]


def _make_agg_deg_kernel():
    out_type = [jax.ShapeDtypeStruct((N_PAD, D), jnp.float32),
                jax.ShapeDtypeStruct((N_PAD, D), jnp.float32)]

    def body(h_hbm, src_hbm, dst_hbm, zagg_hbm, ones_hbm, agg_out, deg_out,
             acc_sh, src_v, dst_v, rows0, rows1, sg0, sg1, ss0, ss1):
        c = lax.axis_index("c")
        s = lax.axis_index("s")

        _zero_stripe(zagg_hbm, acc_sh, s)

        @pl.when(c == 1)
        def _load_ones():
            pltpu.sync_copy(ones_hbm, rows0)

        plsc.subcore_barrier()

        @pl.when(c == 0)
        def _agg():
            bufs = ((rows0, sg0, ss0), (rows1, sg1, ss1))
            _gather_scatter_groups(h_hbm, src_hbm, dst_hbm, acc_sh,
                                   src_v, dst_v, bufs, s * CPT, N_GRP)

        @pl.when(c == 1)
        def _deg():
            _ones_scatter_groups(dst_hbm, acc_sh, dst_v, rows0, ss0, s)

        plsc.subcore_barrier()

        @pl.when(c == 0)
        def _pub_agg():
            _publish_stripe(acc_sh, agg_out, s)

        @pl.when(c == 1)
        def _pub_deg():
            _publish_stripe(acc_sh, deg_out, s)

    return functools.partial(pl.kernel, mesh=_mesh, out_type=out_type,
                             scratch_types=_common_scratch())(body)


CPT0 = 136
CPT1 = CPT - CPT0


def _make_agg_kernel():
    out_type = [jax.ShapeDtypeStruct((NC, N_PAD, D), jnp.float32)]

    def body(h_hbm, src_hbm, dst_hbm, zagg_hbm, agg_out,
             acc_sh, src_v, dst_v, rows0, rows1, sg0, sg1, ss0, ss1):
        c = lax.axis_index("c")
        s = lax.axis_index("s")
        tile_base = jnp.where(c == 0, s * CPT0, NS * CPT0 + s * CPT1)
        n_grp = jnp.where(c == 0, CPT0 // GRP, CPT1 // GRP)

        _zero_stripe(zagg_hbm, acc_sh, s)
        plsc.subcore_barrier()

        bufs = ((rows0, sg0, ss0), (rows1, sg1, ss1))
        _gather_scatter_groups(h_hbm, src_hbm, dst_hbm, acc_sh,
                               src_v, dst_v, bufs, tile_base, n_grp)

        plsc.subcore_barrier()
        pltpu.sync_copy(acc_sh.at[pl.ds(s * RPT, RPT)],
                        agg_out.at[c, pl.ds(s * RPT, RPT)])

    return functools.partial(pl.kernel, mesh=_mesh, out_type=out_type,
                             scratch_types=_common_scratch())(body)


_agg_deg_kernel = _make_agg_deg_kernel()
_agg_kernel = _make_agg_kernel()


BR = 400
GRID = N_NODES // BR


def _affine(agg_ref, deg_ref, h, wm, ws, b):
    deg = jnp.maximum(deg_ref[...], 1.0)
    agg = agg_ref[...] / deg
    return jnp.maximum(
        jnp.dot(agg, wm[...], preferred_element_type=jnp.float32)
        + jnp.dot(h[...], ws[...], preferred_element_type=jnp.float32)
        + b[...], 0.0)


def _dense_body(agg, deg, h, wm, ws, b, out):
    out[...] = _affine(agg, deg, h, wm, ws, b)


def _affine2(aggp, deg_ref, h, wm, ws, b):
    deg = jnp.maximum(deg_ref[...], 1.0)
    agg = (aggp[0] + aggp[1]) / deg
    return jnp.maximum(
        jnp.dot(agg, wm[...], preferred_element_type=jnp.float32)
        + jnp.dot(h[...], ws[...], preferred_element_type=jnp.float32)
        + b[...], 0.0)


def _dense_pool_body(aggp, deg, h, wm, ws, b, out):
    i = pl.program_id(0)
    hn = _affine2(aggp, deg, h, wm, ws, b)
    part = jnp.sum(hn, axis=0, keepdims=True) * (1.0 / N_NODES)

    @pl.when(i == 0)
    def _init():
        out[...] = part

    @pl.when(i != 0)
    def _acc():
        out[...] = out[...] + part


_dense_specs = dict(
    grid=(GRID,),
    in_specs=[
        pl.BlockSpec((BR, D), lambda i: (i, 0)),
        pl.BlockSpec((BR, D), lambda i: (i, 0)),
        pl.BlockSpec((BR, D), lambda i: (i, 0)),
        pl.BlockSpec((D, D), lambda i: (0, 0)),
        pl.BlockSpec((D, D), lambda i: (0, 0)),
        pl.BlockSpec((1, D), lambda i: (0, 0)),
    ],
    compiler_params=pltpu.CompilerParams(
        dimension_semantics=("arbitrary",)),
)

_dense_layer = pl.pallas_call(
    _dense_body,
    out_shape=jax.ShapeDtypeStruct((N_NODES, D), jnp.float32),
    out_specs=pl.BlockSpec((BR, D), lambda i: (i, 0)),
    **_dense_specs,
)

_dense_pool_specs = dict(_dense_specs)
_dense_pool_specs["in_specs"] = (
    [pl.BlockSpec((NC, BR, D), lambda i: (0, i, 0))]
    + list(_dense_specs["in_specs"][1:]))

_dense_pool_layer = pl.pallas_call(
    _dense_pool_body,
    out_shape=jax.ShapeDtypeStruct((1, D), jnp.float32),
    out_specs=pl.BlockSpec((1, D), lambda i: (0, 0)),
    **_dense_pool_specs,
)


def kernel(x, edge_index, W_msg1, W_self1, b1, W_msg2, W_self2, b2):
    src = edge_index[0].astype(jnp.int32)
    dst = edge_index[1].astype(jnp.int32)
    pad = E_PAD - N_EDGES
    pad_dst = N_NODES + (jnp.arange(pad, dtype=jnp.int32) % (N_PAD - N_NODES))
    srcp = jnp.concatenate([src, jnp.zeros((pad,), jnp.int32)])
    dstp = jnp.concatenate([dst, pad_dst])
    srcp = srcp.reshape(E_PAD // CHUNK, CHUNK)
    dstp = dstp.reshape(E_PAD // CHUNK, CHUNK)
    zagg = jnp.zeros((N_PAD, D), jnp.float32)
    ones = jnp.ones((CHUNK, D), jnp.float32)
    b1r = b1.reshape(1, D)
    b2r = b2.reshape(1, D)

    agg1, deg = _agg_deg_kernel(x, srcp, dstp, zagg, ones)
    h1 = _dense_layer(agg1, deg, x, W_msg1, W_self1, b1r)
    (agg2,) = _agg_kernel(h1, srcp, dstp, zagg)
    out = _dense_pool_layer(agg2, deg, h1, W_msg2, W_self2, b2r)
    return out


# speedup vs baseline: 1.5738x; 1.0333x over previous
import functools

import jax
import jax.numpy as jnp
from jax import lax
from jax.experimental import pallas as pl
from jax.experimental.pallas import tpu as pltpu
from jax.experimental.pallas import tpu_sc as plsc

N_NODES = 10000
N_EDGES = 320000
D = 128

NC = 2
NS = 16

CHUNK = 128
CPT = 160
GRP = 8
N_GRP = CPT // GRP
E_PAD = CHUNK * NS * CPT
N_PAD = 10112
RPT = N_PAD // NS

_mesh = plsc.VectorSubcoreMesh(core_axis_name="c", subcore_axis_name="s")


def _zero_stripe(zagg_hbm, sh, s):
    pltpu.sync_copy(zagg_hbm.at[pl.ds(s * RPT, RPT)],
                    sh.at[pl.ds(s * RPT, RPT)])


def _publish_stripe(sh, out, s):
    pltpu.sync_copy(sh.at[pl.ds(s * RPT, RPT)], out.at[pl.ds(s * RPT, RPT)])


def _gather_scatter_groups(h_hbm, src_hbm, dst_hbm, acc_sh,
                           src_v, dst_v, bufs, tile_base, n_grp):
    def grp_body(g, carry):
        base = tile_base + g * GRP
        pltpu.sync_copy(src_hbm.at[pl.ds(base, GRP)], src_v)
        pltpu.sync_copy(dst_hbm.at[pl.ds(base, GRP)], dst_v)

        for b in range(2):
            rows, sg, _ = bufs[b]
            pltpu.async_copy(h_hbm.at[src_v.at[b]], rows, sg)

        for j in range(GRP):
            rows, sg, ss = bufs[j % 2]
            pltpu.make_async_copy(h_hbm.at[src_v.at[j]], rows, sg).wait()
            pltpu.async_copy(rows, acc_sh.at[dst_v.at[j]], ss, add=True)
            if j + 2 < GRP:
                pltpu.make_async_copy(rows, acc_sh.at[dst_v.at[j]],
                                      ss).wait()
                pltpu.async_copy(h_hbm.at[src_v.at[j + 2]], rows, sg)

        for b in range(2):
            rows, _, ss = bufs[b]
            pltpu.make_async_copy(rows, acc_sh.at[dst_v.at[b]], ss).wait()
        return carry

    lax.fori_loop(0, n_grp, grp_body, 0)


def _ones_scatter_groups(dst_hbm, acc_sh, dst_v, ones_v, ss, s):
    def grp_body(g, carry):
        pltpu.sync_copy(dst_hbm.at[pl.ds(s * CPT + g * GRP, GRP)], dst_v)
        for j in range(GRP):
            pltpu.async_copy(ones_v, acc_sh.at[dst_v.at[j]], ss, add=True)
        for j in range(GRP):
            pltpu.make_async_copy(ones_v, acc_sh.at[dst_v.at[j]], ss).wait()
        return carry

    lax.fori_loop(0, N_GRP, grp_body, 0)


def _common_scratch():
    return [
        pltpu.VMEM_SHARED((N_PAD, D), jnp.float32),
        pltpu.VMEM((GRP, CHUNK), jnp.int32),
        pltpu.VMEM((GRP, CHUNK), jnp.int32),
        pltpu.VMEM((CHUNK, D), jnp.float32),
        pltpu.VMEM((CHUNK, D), jnp.float32),
        pltpu.SemaphoreType.DMA,
        pltpu.SemaphoreType.DMA,
        pltpu.SemaphoreType.DMA,
        pltpu.SemaphoreType.DMA,
    ]


def _make_agg_deg_kernel():
    out_type = [jax.ShapeDtypeStruct((N_PAD, D), jnp.float32),
                jax.ShapeDtypeStruct((N_PAD, D), jnp.float32)]

    def body(h_hbm, src_hbm, dst_hbm, zagg_hbm, ones_hbm, agg_out, deg_out,
             acc_sh, src_v, dst_v, rows0, rows1, sg0, sg1, ss0, ss1):
        c = lax.axis_index("c")
        s = lax.axis_index("s")

        _zero_stripe(zagg_hbm, acc_sh, s)

        @pl.when(c == 1)
        def _load_ones():
            pltpu.sync_copy(ones_hbm, rows0)

        plsc.subcore_barrier()

        @pl.when(c == 0)
        def _agg():
            bufs = ((rows0, sg0, ss0), (rows1, sg1, ss1))
            _gather_scatter_groups(h_hbm, src_hbm, dst_hbm, acc_sh,
                                   src_v, dst_v, bufs, s * CPT, N_GRP)

        @pl.when(c == 1)
        def _deg():
            _ones_scatter_groups(dst_hbm, acc_sh, dst_v, rows0, ss0, s)

        plsc.subcore_barrier()

        @pl.when(c == 0)
        def _pub_agg():
            _publish_stripe(acc_sh, agg_out, s)

        @pl.when(c == 1)
        def _pub_deg():
            _publish_stripe(acc_sh, deg_out, s)

    return functools.partial(pl.kernel, mesh=_mesh, out_type=out_type,
                             scratch_types=_common_scratch())(body)


CPT0 = 144
CPT1 = CPT - CPT0


def _make_agg_kernel():
    out_type = [jax.ShapeDtypeStruct((NC, N_PAD, D), jnp.float32)]

    def body(h_hbm, src_hbm, dst_hbm, zagg_hbm, agg_out,
             acc_sh, src_v, dst_v, rows0, rows1, sg0, sg1, ss0, ss1):
        c = lax.axis_index("c")
        s = lax.axis_index("s")
        tile_base = jnp.where(c == 0, s * CPT0, NS * CPT0 + s * CPT1)
        n_grp = jnp.where(c == 0, CPT0 // GRP, CPT1 // GRP)

        _zero_stripe(zagg_hbm, acc_sh, s)
        plsc.subcore_barrier()

        bufs = ((rows0, sg0, ss0), (rows1, sg1, ss1))
        _gather_scatter_groups(h_hbm, src_hbm, dst_hbm, acc_sh,
                               src_v, dst_v, bufs, tile_base, n_grp)

        plsc.subcore_barrier()
        pltpu.sync_copy(acc_sh.at[pl.ds(s * RPT, RPT)],
                        agg_out.at[c, pl.ds(s * RPT, RPT)])

    return functools.partial(pl.kernel, mesh=_mesh, out_type=out_type,
                             scratch_types=_common_scratch())(body)


_agg_deg_kernel = _make_agg_deg_kernel()
_agg_kernel = _make_agg_kernel()


BR = 400
GRID = N_NODES // BR


def _affine(agg_ref, deg_ref, h, wm, ws, b):
    deg = jnp.maximum(deg_ref[...], 1.0)
    agg = agg_ref[...] / deg
    return jnp.maximum(
        jnp.dot(agg, wm[...], preferred_element_type=jnp.float32)
        + jnp.dot(h[...], ws[...], preferred_element_type=jnp.float32)
        + b[...], 0.0)


def _dense_body(agg, deg, h, wm, ws, b, out):
    out[...] = _affine(agg, deg, h, wm, ws, b)


def _affine2(aggp, deg_ref, h, wm, ws, b):
    deg = jnp.maximum(deg_ref[...], 1.0)
    agg = (aggp[0] + aggp[1]) / deg
    return jnp.maximum(
        jnp.dot(agg, wm[...], preferred_element_type=jnp.float32)
        + jnp.dot(h[...], ws[...], preferred_element_type=jnp.float32)
        + b[...], 0.0)


def _dense_pool_body(aggp, deg, h, wm, ws, b, out):
    i = pl.program_id(0)
    hn = _affine2(aggp, deg, h, wm, ws, b)
    part = jnp.sum(hn, axis=0, keepdims=True) * (1.0 / N_NODES)

    @pl.when(i == 0)
    def _init():
        out[...] = part

    @pl.when(i != 0)
    def _acc():
        out[...] = out[...] + part


_dense_specs = dict(
    grid=(GRID,),
    in_specs=[
        pl.BlockSpec((BR, D), lambda i: (i, 0)),
        pl.BlockSpec((BR, D), lambda i: (i, 0)),
        pl.BlockSpec((BR, D), lambda i: (i, 0)),
        pl.BlockSpec((D, D), lambda i: (0, 0)),
        pl.BlockSpec((D, D), lambda i: (0, 0)),
        pl.BlockSpec((1, D), lambda i: (0, 0)),
    ],
    compiler_params=pltpu.CompilerParams(
        dimension_semantics=("arbitrary",)),
)

_dense_layer = pl.pallas_call(
    _dense_body,
    out_shape=jax.ShapeDtypeStruct((N_NODES, D), jnp.float32),
    out_specs=pl.BlockSpec((BR, D), lambda i: (i, 0)),
    **_dense_specs,
)

_dense_pool_specs = dict(_dense_specs)
_dense_pool_specs["in_specs"] = (
    [pl.BlockSpec((NC, BR, D), lambda i: (0, i, 0))]
    + list(_dense_specs["in_specs"][1:]))

_dense_pool_layer = pl.pallas_call(
    _dense_pool_body,
    out_shape=jax.ShapeDtypeStruct((1, D), jnp.float32),
    out_specs=pl.BlockSpec((1, D), lambda i: (0, 0)),
    **_dense_pool_specs,
)


def kernel(x, edge_index, W_msg1, W_self1, b1, W_msg2, W_self2, b2):
    src = edge_index[0].astype(jnp.int32)
    dst = edge_index[1].astype(jnp.int32)
    pad = E_PAD - N_EDGES
    pad_dst = N_NODES + (jnp.arange(pad, dtype=jnp.int32) % (N_PAD - N_NODES))
    srcp = jnp.concatenate([src, jnp.zeros((pad,), jnp.int32)])
    dstp = jnp.concatenate([dst, pad_dst])
    srcp = srcp.reshape(E_PAD // CHUNK, CHUNK)
    dstp = dstp.reshape(E_PAD // CHUNK, CHUNK)
    zagg = jnp.zeros((N_PAD, D), jnp.float32)
    ones = jnp.ones((CHUNK, D), jnp.float32)
    b1r = b1.reshape(1, D)
    b2r = b2.reshape(1, D)

    agg1, deg = _agg_deg_kernel(x, srcp, dstp, zagg, ones)
    h1 = _dense_layer(agg1, deg, x, W_msg1, W_self1, b1r)
    (agg2,) = _agg_kernel(h1, srcp, dstp, zagg)
    out = _dense_pool_layer(agg2, deg, h1, W_msg2, W_self2, b2r)
    return out
